# baseline (speedup 1.0000x reference)
import functools

import jax
import jax.numpy as jnp
from jax import lax
from jax.experimental import pallas as pl
from jax.experimental.pallas import tpu as pltpu
from jax.experimental.pallas import tpu_sc as plsc

P = 100000
B = 4096
T = 200
C = 32

NC = 2
NS = 16
NW = NC * NS
BPW = B // NW
TC_ = 10
NCHUNK = T // TC_
TILE_W = C * BPW

EXT = 102400
RT = EXT // 128
RTPW = RT // NS

_mesh = plsc.VectorSubcoreMesh(core_axis_name="c", subcore_axis_name="s")


@functools.partial(
    pl.kernel,
    mesh=_mesh,
    out_type=[
        jax.ShapeDtypeStruct((T, B * C), jnp.float32),
        jax.ShapeDtypeStruct((NC, EXT, C), jnp.float32),
        jax.ShapeDtypeStruct((TILE_W,), jnp.float32),
        jax.ShapeDtypeStruct((BPW, C), jnp.float32),
    ],
    scratch_types=[
        pltpu.VMEM((BPW,), jnp.int32),
        pltpu.VMEM((16,), jnp.int32),
        pltpu.SMEM((BPW,), jnp.int32),
        [pltpu.VMEM((4, 8, 128), jnp.float32) for _ in range(2)],
        [pltpu.VMEM((BPW, C + 1), jnp.float32) for _ in range(2)],
        [pltpu.VMEM((BPW * TC_, C), jnp.float32) for _ in range(2)],
        [pltpu.VMEM((TILE_W,), jnp.float32) for _ in range(2)],
        [pltpu.SemaphoreType.DMA for _ in range(2)],
        [pltpu.SemaphoreType.DMA for _ in range(2)],
        [pltpu.SemaphoreType.DMA for _ in range(2)],
        [pltpu.SemaphoreType.DMA for _ in range(2)],
    ],
    compiler_params=pltpu.CompilerParams(
        needs_layout_passes=False, use_tc_tiling_on_sc=False
    ),
)
def _sc_gather(idx_hbm, shift_hbm, dataq_hbm,
               out_hbm, tbl_hbm, sink1_hbm, sink2_hbm,
               idx_v, shift_v, idx_s, bbs, rowbufs, stages, tiles,
               bsems, vsems, rsems, wsems):
    cid = lax.axis_index("c")
    sid = lax.axis_index("s")
    wid = sid * NC + cid

    iota = lax.iota(jnp.int32, 16)

    def fire_blocks(q, p):
        rt = sid * RTPW + jnp.minimum(q, RTPW - 1)
        for ct in range(4):
            pltpu.async_copy(
                dataq_hbm.at[ct * RT + rt], bbs[p].at[ct], bsems[p]
            )

    def wait_blocks(p):
        pltpu.make_async_copy(
            dataq_hbm.at[pl.ds(0, 4)], bbs[p], bsems[p]
        ).wait()

    def wait_rowbuf(p):
        pltpu.make_async_copy(
            rowbufs[p].at[:, pl.ds(0, C)],
            tbl_hbm.at[cid, pl.ds(0, BPW)],
            vsems[p],
        ).wait()

    lvec = [iota + lg * 16 for lg in range(8)]

    def detile_q(q, p):
        rt = sid * RTPW + q
        for ct in range(4):
            for s in range(8):
                cvec = jnp.full((16,), ct * 8 + s, jnp.int32)
                for lg in range(8):
                    val = bbs[p][ct, s, pl.ds(lg * 16, 16)]
                    plsc.store_scatter(
                        rowbufs[p], [lvec[lg], cvec], val
                    )
        pltpu.async_copy(
            rowbufs[p].at[:, pl.ds(0, C)],
            tbl_hbm.at[cid, pl.ds(rt * 128, BPW)],
            vsems[p],
        )

    pltpu.async_copy(rowbufs[0].at[:, pl.ds(0, C)], sink2_hbm, vsems[0])
    pltpu.async_copy(rowbufs[1].at[:, pl.ds(0, C)], sink2_hbm, vsems[1])
    fire_blocks(0, 0)

    def detile_pair(qp, carry):
        for k in range(2):
            q = 2 * qp + k
            fire_blocks(q + 1, (k + 1) % 2)
            wait_blocks(k)
            wait_rowbuf(k)
            detile_q(q, k)
        return carry

    lax.fori_loop(0, RTPW // 2, detile_pair, 0)

    wait_blocks(0)
    wait_rowbuf(0)
    wait_rowbuf(1)
    plsc.subcore_barrier()

    base_b = wid * BPW
    pltpu.sync_copy(idx_hbm.at[pl.ds(base_b, BPW)], idx_v)
    pltpu.sync_copy(shift_hbm, shift_v)
    shift_vec = shift_v[...]

    for g in range(BPW // 16):
        v = idx_v[pl.ds(g * 16, 16)] + shift_vec
        v = jnp.where(v >= P, v - P, v)
        v = jnp.where(v < 0, v + P, v)
        for k in range(16):
            idx_s[g * 16 + k] = v[k]

    rbase = [((iota + d) % 16) * TC_ for d in range(16)]
    wbase = [iota * BPW + (iota + d) % 16 for d in range(16)]
    colv = [iota, iota + 16]

    def fire_reads(g, sb):
        t0 = jnp.minimum(g, NCHUNK - 1) * TC_

        def one(l, carry):
            start = idx_s[l] + t0
            pltpu.async_copy(
                tbl_hbm.at[cid, pl.ds(start, TC_)],
                stages[sb].at[pl.ds(l * TC_, TC_)],
                rsems[sb],
            )
            return carry

        lax.fori_loop(0, BPW, one, 0)

    def wait_reads(sb):
        pltpu.make_async_copy(
            tbl_hbm.at[cid, pl.ds(0, BPW * TC_)], stages[sb], rsems[sb]
        ).wait()

    def wait_tile(tb):
        pltpu.make_async_copy(sink1_hbm, tiles[tb], wsems[tb]).wait()

    def fire_writes(t, tb):
        for ct in range(4):
            pltpu.async_copy(
                tiles[tb].at[pl.ds(ct * 8 * BPW, 8 * BPW)],
                out_hbm.at[t, pl.ds((ct * NW + wid) * 8 * BPW, 8 * BPW)],
                wsems[tb],
            )

    def transpose_t(stage, tile, trel):
        def per_lb(lb, carry):
            groups = []
            for cb in range(2):
                rs = lb * 16 * TC_ + trel
                ws = cb * 16 * BPW + lb * 16
                for h in range(2):
                    groups.append((rs, ws, cb, h * 8))
            rs0, ws0, cb0, d00 = groups[0]
            prev = [
                plsc.load_gather(stage, [rbase[d00 + k] + rs0, colv[cb0]])
                for k in range(8)
            ]
            prev_ws, prev_d0 = ws0, d00
            for rs, ws, cb, d0 in groups[1:]:
                cur = []
                for k in range(8):
                    cur.append(
                        plsc.load_gather(
                            stage, [rbase[d0 + k] + rs, colv[cb]]
                        )
                    )
                    plsc.store_scatter(
                        tile, [wbase[prev_d0 + k] + prev_ws], prev[k]
                    )
                prev, prev_ws, prev_d0 = cur, ws, d0
            for k in range(8):
                plsc.store_scatter(
                    tile, [wbase[prev_d0 + k] + prev_ws], prev[k]
                )
            return carry

        lax.fori_loop(0, BPW // 16, per_lb, 0)

    pltpu.async_copy(tiles[0], sink1_hbm, wsems[0])
    pltpu.async_copy(tiles[1], sink1_hbm, wsems[1])

    fire_reads(0, 0)
    fire_reads(1, 1)

    def chunk_pair(gp, carry):
        for sb in range(2):
            g = 2 * gp + sb
            wait_reads(sb)

            def tpair(j, carry2):
                for tb in range(2):
                    trel = 2 * j + tb
                    wait_tile(tb)
                    transpose_t(stages[sb], tiles[tb], trel)
                    fire_writes(g * TC_ + trel, tb)
                return carry2

            lax.fori_loop(0, TC_ // 2, tpair, 0)
            fire_reads(g + 2, sb)
        return carry

    lax.fori_loop(0, NCHUNK // 2, chunk_pair, 0)

    wait_reads(0)
    wait_reads(1)
    wait_tile(0)
    wait_tile(1)


def kernel(index, length, data):
    shift = jnp.broadcast_to(
        (jnp.asarray(length, jnp.int32) - T).reshape(()), (16,)
    ).astype(jnp.int32)
    data_ext = jnp.concatenate([data, data[: EXT - P]], axis=0)
    data_q = (
        data_ext.T.reshape(4, 8, RT, 128).transpose(0, 2, 1, 3)
    ).reshape(4 * RT, 8, 128)
    out, _, _, _ = _sc_gather(index.astype(jnp.int32), shift, data_q)
    out5 = out.reshape(T, 4, NW, 8, BPW)
    return jnp.transpose(out5, (2, 4, 0, 1, 3)).reshape(B, T, C)


# speedup vs baseline: 1.2869x; 1.2869x over previous
import functools

import jax
import jax.numpy as jnp
from jax import lax
from jax.experimental import pallas as pl
from jax.experimental.pallas import tpu as pltpu
from jax.experimental.pallas import tpu_sc as plsc

P = 100000
B = 4096
T = 200
C = 32

NC = 2
NS = 16
NW = NC * NS
BPW = B // NW
TC_ = 10
NCHUNK = T // TC_
TILE_W = C * BPW

_mesh = plsc.VectorSubcoreMesh(core_axis_name="c", subcore_axis_name="s")


@functools.partial(
    pl.kernel,
    mesh=_mesh,
    out_type=[
        jax.ShapeDtypeStruct((T, B * C), jnp.float32),
        jax.ShapeDtypeStruct((TILE_W,), jnp.float32),
    ],
    scratch_types=[
        pltpu.VMEM((BPW,), jnp.int32),
        pltpu.VMEM((16,), jnp.int32),
        pltpu.SMEM((BPW,), jnp.int32),
        [pltpu.VMEM((BPW * TC_ * C,), jnp.float32) for _ in range(2)],
        [pltpu.VMEM((TILE_W,), jnp.float32) for _ in range(2)],
        [pltpu.SemaphoreType.DMA for _ in range(2)],
        [pltpu.SemaphoreType.DMA for _ in range(2)],
    ],
    compiler_params=pltpu.CompilerParams(
        needs_layout_passes=False, use_tc_tiling_on_sc=False
    ),
)
def _sc_gather(idx_hbm, shift_hbm, data_hbm, out_hbm, dummy_hbm,
               idx_v, shift_v, idx_s, stages, tiles, rsems, wsems):
    wid = lax.axis_index("s") * NC + lax.axis_index("c")
    base_b = wid * BPW

    pltpu.sync_copy(idx_hbm.at[pl.ds(base_b, BPW)], idx_v)
    pltpu.sync_copy(shift_hbm, shift_v)
    shift_vec = shift_v[...]

    for g in range(BPW // 16):
        v = idx_v[pl.ds(g * 16, 16)] + shift_vec
        v = jnp.where(v >= P, v - P, v)
        v = jnp.where(v < 0, v + P, v)
        for k in range(16):
            idx_s[g * 16 + k] = v[k]

    iota = lax.iota(jnp.int32, 16)
    rbase = [((iota + d) % 16) * (TC_ * C) + iota for d in range(16)]
    wbase = [iota * BPW + (iota + d) % 16 for d in range(16)]

    def fire_reads(g, sb):
        t0 = jnp.minimum(g, NCHUNK - 1) * TC_

        def one(l, carry):
            start = (idx_s[l] + t0) * C
            pltpu.async_copy(
                data_hbm.at[pl.ds(start, TC_ * C)],
                stages[sb].at[pl.ds(l * TC_ * C, TC_ * C)],
                rsems[sb],
            )
            return carry

        lax.fori_loop(0, BPW, one, 0)

    def wait_reads(sb):
        pltpu.make_async_copy(
            data_hbm.at[pl.ds(0, BPW * TC_ * C)], stages[sb], rsems[sb]
        ).wait()

    def wait_tile(tb):
        pltpu.make_async_copy(dummy_hbm, tiles[tb], wsems[tb]).wait()

    def fire_writes(t, tb):
        for ct in range(4):
            pltpu.async_copy(
                tiles[tb].at[pl.ds(ct * 8 * BPW, 8 * BPW)],
                out_hbm.at[t, pl.ds((ct * NW + wid) * 8 * BPW, 8 * BPW)],
                wsems[tb],
            )

    def transpose_t(stage, tile, trel):
        def per_lb(lb, carry):
            groups = []
            for cb in range(2):
                rs = lb * 16 * TC_ * C + trel * C + cb * 16
                ws = cb * 16 * BPW + lb * 16
                for h in range(2):
                    groups.append((rs, ws, h * 8))
            rs0, ws0, d00 = groups[0]
            prev = [
                plsc.load_gather(stage, [rbase[d00 + k] + rs0])
                for k in range(8)
            ]
            prev_ws, prev_d0 = ws0, d00
            for rs, ws, d0 in groups[1:]:
                cur = []
                for k in range(8):
                    cur.append(
                        plsc.load_gather(stage, [rbase[d0 + k] + rs])
                    )
                    plsc.store_scatter(
                        tile, [wbase[prev_d0 + k] + prev_ws], prev[k]
                    )
                prev, prev_ws, prev_d0 = cur, ws, d0
            for k in range(8):
                plsc.store_scatter(
                    tile, [wbase[prev_d0 + k] + prev_ws], prev[k]
                )
            return carry

        lax.fori_loop(0, BPW // 16, per_lb, 0)

    pltpu.async_copy(tiles[0], dummy_hbm, wsems[0])
    pltpu.async_copy(tiles[1], dummy_hbm, wsems[1])

    fire_reads(0, 0)
    fire_reads(1, 1)

    def chunk_pair(gp, carry):
        for sb in range(2):
            g = 2 * gp + sb
            wait_reads(sb)

            def tpair(j, carry2):
                for tb in range(2):
                    trel = 2 * j + tb
                    wait_tile(tb)
                    transpose_t(stages[sb], tiles[tb], trel)
                    fire_writes(g * TC_ + trel, tb)
                return carry2

            lax.fori_loop(0, TC_ // 2, tpair, 0)
            fire_reads(g + 2, sb)
        return carry

    lax.fori_loop(0, NCHUNK // 2, chunk_pair, 0)

    wait_reads(0)
    wait_reads(1)
    wait_tile(0)
    wait_tile(1)


def kernel(index, length, data):
    shift = jnp.broadcast_to(
        (jnp.asarray(length, jnp.int32) - T).reshape(()), (16,)
    ).astype(jnp.int32)
    data_flat = data.reshape(-1)
    data_ext = jnp.concatenate([data_flat, data_flat[: T * C]])
    out, _ = _sc_gather(index.astype(jnp.int32), shift, data_ext)
    out5 = out.reshape(T, 4, NW, 8, BPW)
    return jnp.transpose(out5, (2, 4, 0, 1, 3)).reshape(B, T, C)
